# Initial kernel scaffold; baseline (speedup 1.0000x reference)
import jax
import jax.numpy as jnp
from jax.experimental import pallas as pl


def kernel(s, edge_index, W1_0, b1_0, W2_0, b2_0, a_0, W1_1, b1_1, W2_1, b2_1, a_1, W1_2, b1_2, W2_2, b2_2, a_2):
    raise NotImplementedError("write your pallas kernel here")


# speedup vs baseline: 4.1412x; 4.1412x over previous
import functools

import jax
import jax.numpy as jnp
from jax import lax
from jax.experimental import pallas as pl
from jax.experimental.pallas import tpu as pltpu, tpu_sc as plsc

f32 = jnp.float32
i32 = jnp.int32
bf16 = jnp.bfloat16

N = 10000
E = 320000
D = 128
L = 16
NC = 2
NS = 16
NW = NC * NS
PN = 10240
CHN = PN // NS

EC_S = 2000
E_TILE_S = E // NS
NCH_S = E_TILE_S // EC_S
NVR_S = EC_S // L

ECM = 80
E_HALF = E // NC
E_TILE_M = E_HALF // NS
NCH_M = E_TILE_M // ECM

E_TILE_D = E // NW
NCH_D = E_TILE_D // ECM

RSELF = 40

BR = 2000
BE = 4000

_mesh = plsc.VectorSubcoreMesh(core_axis_name="c", subcore_axis_name="s")
_CP = pltpu.CompilerParams(needs_layout_passes=False)


def _delta_body(src_hbm, dst_hbm, s2_hbm, delta_hbm,
                es_m, ed_m, rows_d, rows_s, semd, sems):
    cid = lax.axis_index("c")
    sid = lax.axis_index("s")
    base = (cid * NS + sid) * E_TILE_D

    def _chunk(c, carry):
        off = base + c * ECM
        pltpu.sync_copy(src_hbm.at[pl.ds(off, ECM)], es_m)
        pltpu.sync_copy(dst_hbm.at[pl.ds(off, ECM)], ed_m)
        cpd = pltpu.async_copy(s2_hbm.at[ed_m], rows_d, semd)
        cps = pltpu.async_copy(s2_hbm.at[es_m], rows_s, sems)
        cpd.wait()
        cps.wait()
        for r in range(ECM):
            for cc in range(D // L):
                sl = pl.ds(cc * L, L)
                rows_d[r, sl] = rows_d[r, sl] - rows_s[r, sl]
        pltpu.sync_copy(rows_d, delta_hbm.at[pl.ds(off, ECM)])
        return carry

    lax.fori_loop(0, NCH_D, _chunk, 0)


_sc_delta = functools.partial(
    pl.kernel,
    out_type=jax.ShapeDtypeStruct((E, D), f32),
    mesh=_mesh,
    compiler_params=_CP,
    scratch_types=[
        pltpu.VMEM((ECM,), i32),
        pltpu.VMEM((ECM,), i32),
        pltpu.VMEM((ECM, D), f32),
        pltpu.VMEM((ECM, D), f32),
        pltpu.SemaphoreType.DMA,
        pltpu.SemaphoreType.DMA,
    ],
)(_delta_body)


def _sc_body(src_hbm, dst_hbm, al_hbm, s2_hbm, agg_hbm, part_hbm,
             m_v, den_v, af_v, ed_v, acc_v, tmp_v,
             ds_s, vs_s, es_m, ed_m, ea_m, coef_v, rows_v, cs_v,
             red_sh, agg_sh, sem):
    cid = lax.axis_index("c")
    sid = lax.axis_index("s")
    iot = lax.iota(i32, L)
    zero16 = jnp.zeros((L,), f32)

    def _zm(k, carry):
        m_v[pl.ds(k * L, L)] = zero16
        den_v[pl.ds(k * L, L)] = zero16
        return carry

    lax.fori_loop(0, PN // L, _zm, 0)

    def _seg_combine(dv, val, op):
        sk, sv = plsc.sort_key_val(dv, val)
        ds_s[...] = sk
        vs_s[...] = sv
        v = sv
        for sh in (1, 2, 4, 8):
            jj = jnp.maximum(iot - sh, 0)
            pv = plsc.load_gather(vs_s, [jj])
            pd = plsc.load_gather(ds_s, [jj])
            take = (pd == sk) & (iot >= sh)
            v = jnp.where(take, op(v, pv), v)
            vs_s[...] = v
        nd = plsc.load_gather(ds_s, [jnp.minimum(iot + 1, L - 1)])
        last = (nd != sk) | (iot == L - 1)
        return sk, v, last

    ebase = sid * E_TILE_S

    def _max_chunk(c, carry):
        off = ebase + c * EC_S
        pltpu.sync_copy(al_hbm.at[pl.ds(off, EC_S)], af_v)
        pltpu.sync_copy(dst_hbm.at[pl.ds(off, EC_S)], ed_v)

        def _vr(j, carry2):
            av = af_v[pl.ds(j * L, L)]
            dv = ed_v[pl.ds(j * L, L)]
            sk, v, last = _seg_combine(dv, av, jnp.maximum)
            cur = plsc.load_gather(m_v, [sk])
            plsc.store_scatter(m_v, [sk], jnp.maximum(cur, v), mask=last)
            return carry2

        lax.fori_loop(0, NVR_S, _vr, 0)
        return carry

    lax.fori_loop(0, NCH_S, _max_chunk, 0)

    nbase = sid * CHN
    pltpu.sync_copy(m_v, part_hbm.at[cid, sid])
    plsc.subcore_barrier()
    pltpu.sync_copy(part_hbm.at[cid, 0, pl.ds(nbase, CHN)], acc_v)
    for k in range(1, NS):
        pltpu.sync_copy(part_hbm.at[cid, k, pl.ds(nbase, CHN)], tmp_v)

        def _redm(q, carry):
            acc_v[pl.ds(q * L, L)] = jnp.maximum(acc_v[pl.ds(q * L, L)],
                                                 tmp_v[pl.ds(q * L, L)])
            return carry

        lax.fori_loop(0, CHN // L, _redm, 0)
    pltpu.sync_copy(acc_v, red_sh.at[pl.ds(nbase, CHN)])
    plsc.subcore_barrier()
    pltpu.sync_copy(red_sh, m_v)

    def _den_chunk(c, carry):
        off = ebase + c * EC_S
        pltpu.sync_copy(al_hbm.at[pl.ds(off, EC_S)], af_v)
        pltpu.sync_copy(dst_hbm.at[pl.ds(off, EC_S)], ed_v)

        def _vr(j, carry2):
            av = af_v[pl.ds(j * L, L)]
            dv = ed_v[pl.ds(j * L, L)]
            w = jnp.exp(av - plsc.load_gather(m_v, [dv]))
            sk, v, last = _seg_combine(dv, w, jnp.add)
            cur = plsc.load_gather(den_v, [sk])
            plsc.store_scatter(den_v, [sk], cur + v, mask=last)
            return carry2

        lax.fori_loop(0, NVR_S, _vr, 0)
        return carry

    lax.fori_loop(0, NCH_S, _den_chunk, 0)

    pltpu.sync_copy(den_v, part_hbm.at[cid, sid])
    plsc.subcore_barrier()
    pltpu.sync_copy(part_hbm.at[cid, 0, pl.ds(nbase, CHN)], acc_v)
    for k in range(1, NS):
        pltpu.sync_copy(part_hbm.at[cid, k, pl.ds(nbase, CHN)], tmp_v)

        def _redd(q, carry):
            acc_v[pl.ds(q * L, L)] = (acc_v[pl.ds(q * L, L)]
                                      + tmp_v[pl.ds(q * L, L)])
            return carry

        lax.fori_loop(0, CHN // L, _redd, 0)

    def _selfterm(q, carry):
        mm = m_v[pl.ds(nbase + q * L, L)]
        acc_v[pl.ds(q * L, L)] = (acc_v[pl.ds(q * L, L)]
                                  + jnp.exp(-mm) + f32(1e-16))
        return carry

    lax.fori_loop(0, CHN // L, _selfterm, 0)
    pltpu.sync_copy(acc_v, red_sh.at[pl.ds(nbase, CHN)])
    plsc.subcore_barrier()
    pltpu.sync_copy(red_sh, den_v)

    csc = jnp.where(cid == 0, f32(1.0), f32(0.0))
    gbase = sid * CHN
    nself = jnp.where(gbase + CHN > N, (N - gbase) // RSELF, CHN // RSELF)

    def _csq(q, carry):
        mm = m_v[pl.ds(gbase + q * L, L)]
        dd = den_v[pl.ds(gbase + q * L, L)]
        cs_v[pl.ds(q * L, L)] = jnp.exp(-mm) / dd * csc
        return carry

    lax.fori_loop(0, CHN // L, _csq, 0)

    def _sinit(c, carry):
        g0 = gbase + c * RSELF
        pltpu.sync_copy(s2_hbm.at[pl.ds(g0, RSELF)], rows_v.at[pl.ds(0, RSELF)])
        for r in range(RSELF):
            ridx = jnp.broadcast_to(c * RSELF + r, (L,)).astype(i32)
            spl = plsc.load_gather(cs_v, [ridx])
            for cc in range(D // L):
                rows_v[r, pl.ds(cc * L, L)] = rows_v[r, pl.ds(cc * L, L)] * spl
        pltpu.sync_copy(rows_v.at[pl.ds(0, RSELF)], agg_sh.at[pl.ds(g0, RSELF)])
        return carry

    lax.fori_loop(0, nself, _sinit, 0)
    plsc.subcore_barrier()

    mbase = cid * E_HALF + sid * E_TILE_M

    def _msg(c, carry):
        off = mbase + c * ECM
        pltpu.sync_copy(src_hbm.at[pl.ds(off, ECM)], es_m)
        pltpu.sync_copy(dst_hbm.at[pl.ds(off, ECM)], ed_m)
        pltpu.sync_copy(al_hbm.at[pl.ds(off, ECM)], ea_m)

        def _cf(j, carry2):
            av = ea_m[pl.ds(j * L, L)]
            dv = ed_m[pl.ds(j * L, L)]
            w = jnp.exp(av - plsc.load_gather(m_v, [dv]))
            coef_v[pl.ds(j * L, L)] = w / plsc.load_gather(den_v, [dv])
            return carry2

        lax.fori_loop(0, ECM // L, _cf, 0)
        pltpu.async_copy(s2_hbm.at[es_m], rows_v, sem).wait()

        def _scale(r, carry2):
            ridx = jnp.broadcast_to(r, (L,)).astype(i32)
            spl = plsc.load_gather(coef_v, [ridx])
            for cc in range(D // L):
                cidx = cc * L + iot
                v = plsc.load_gather(rows_v, [ridx, cidx])
                plsc.store_scatter(rows_v, [ridx, cidx], v * spl)
            return carry2

        lax.fori_loop(0, ECM, _scale, 0)
        pltpu.sync_copy(rows_v, agg_sh.at[ed_m], add=True)
        return carry

    lax.fori_loop(0, NCH_M, _msg, 0)
    plsc.subcore_barrier()

    pltpu.sync_copy(agg_sh.at[pl.ds(gbase, CHN)],
                    agg_hbm.at[cid, pl.ds(gbase, CHN)])


_sc_layer = functools.partial(
    pl.kernel,
    out_type=(jax.ShapeDtypeStruct((NC, PN, D), f32),
              jax.ShapeDtypeStruct((NC, NS, PN), f32)),
    mesh=_mesh,
    compiler_params=_CP,
    scratch_types=[
        pltpu.VMEM((PN,), f32),
        pltpu.VMEM((PN,), f32),
        pltpu.VMEM((EC_S,), f32),
        pltpu.VMEM((EC_S,), i32),
        pltpu.VMEM((CHN,), f32),
        pltpu.VMEM((CHN,), f32),
        pltpu.VMEM((L,), i32),
        pltpu.VMEM((L,), f32),
        pltpu.VMEM((ECM,), i32),
        pltpu.VMEM((ECM,), i32),
        pltpu.VMEM((ECM,), f32),
        pltpu.VMEM((ECM,), f32),
        pltpu.VMEM((ECM, D), f32),
        pltpu.VMEM((CHN,), f32),
        pltpu.VMEM_SHARED((PN,), f32),
        pltpu.VMEM_SHARED((PN, D), f32),
        pltpu.SemaphoreType.DMA,
    ],
)(_sc_body)


def _s2_body(s_ref, w2_ref, b2_ref, s2_ref):
    s2_ref[...] = lax.dot_general(
        s_ref[...], w2_ref[...], (((1,), (1,)), ((), ())),
        preferred_element_type=f32) + b2_ref[...]


def _tc_s2(s, W2, b2):
    return pl.pallas_call(
        _s2_body,
        grid=(N // BR,),
        in_specs=[pl.BlockSpec((BR, D), lambda i: (i, 0)),
                  pl.BlockSpec((D, D), lambda i: (0, 0)),
                  pl.BlockSpec((1, D), lambda i: (0, 0))],
        out_specs=pl.BlockSpec((BR, D), lambda i: (i, 0)),
        out_shape=jax.ShapeDtypeStruct((N, D), f32),
    )(s, W2, b2.reshape(1, D))


def _al_body(d_ref, a_ref, o_ref):
    db = d_ref[...].astype(bf16)
    ab = a_ref[...].astype(bf16)
    o_ref[...] = lax.dot_general(db, ab, (((1,), (0,)), ((), ())),
                                 preferred_element_type=f32)


def _tc_alpha(delta, a):
    return pl.pallas_call(
        _al_body,
        grid=(E // BE,),
        in_specs=[pl.BlockSpec((BE, D), lambda i: (i, 0)),
                  pl.BlockSpec((D, 1), lambda i: (0, 0))],
        out_specs=pl.BlockSpec((BE, 1), lambda i: (i, 0)),
        out_shape=jax.ShapeDtypeStruct((E, 1), f32),
    )(delta, a)


def _out_body(s_ref, w1_ref, b1_ref, a0_ref, a1_ref, o_ref):
    o = lax.dot_general(s_ref[...], w1_ref[...], (((1,), (1,)), ((), ())),
                        preferred_element_type=f32) + b1_ref[...]
    o = o + a0_ref[0] + a1_ref[0]
    o_ref[...] = jnp.maximum(o, f32(0.0))


def _tc_out(s, W1, b1, agg):
    return pl.pallas_call(
        _out_body,
        grid=(N // BR,),
        in_specs=[pl.BlockSpec((BR, D), lambda i: (i, 0)),
                  pl.BlockSpec((D, D), lambda i: (0, 0)),
                  pl.BlockSpec((1, D), lambda i: (0, 0)),
                  pl.BlockSpec((1, BR, D), lambda i: (0, i, 0)),
                  pl.BlockSpec((1, BR, D), lambda i: (1, i, 0))],
        out_specs=pl.BlockSpec((BR, D), lambda i: (i, 0)),
        out_shape=jax.ShapeDtypeStruct((N, D), f32),
    )(s, W1, b1.reshape(1, D), agg, agg)


def kernel(s, edge_index,
           W1_0, b1_0, W2_0, b2_0, a_0,
           W1_1, b1_1, W2_1, b2_1, a_1,
           W1_2, b1_2, W2_2, b2_2, a_2):
    src = edge_index[0]
    dst = edge_index[1]
    params = [
        (W1_0, b1_0, W2_0, b2_0, a_0),
        (W1_1, b1_1, W2_1, b2_1, a_1),
        (W1_2, b1_2, W2_2, b2_2, a_2),
    ]
    out = s
    for (W1, b1, W2, b2, a) in params:
        s2 = _tc_s2(out, W2, b2)
        delta = _sc_delta(src, dst, s2)
        alpha = _tc_alpha(delta, a)
        agg, _ = _sc_layer(src, dst, alpha.reshape(E), s2)
        out = _tc_out(out, W1, b1, agg)
    return out


# speedup vs baseline: 4.1806x; 1.0095x over previous
import functools

import jax
import jax.numpy as jnp
from jax import lax
from jax.experimental import pallas as pl
from jax.experimental.pallas import tpu as pltpu, tpu_sc as plsc

f32 = jnp.float32
i32 = jnp.int32
bf16 = jnp.bfloat16

N = 10000
E = 320000
D = 128
L = 16
NC = 2
NS = 16
NW = NC * NS
PN = 10240
CHN = PN // NS

EC_S = 2000
E_TILE_S = E // NS
NCH_S = E_TILE_S // EC_S
NVR_S = EC_S // L

ECM = 80
E_HALF = E // NC
E_TILE_M = E_HALF // NS
NCH_M = E_TILE_M // ECM

E_TILE_D = E // NW
ECD = 400
NCH_D = E_TILE_D // ECD

RSELF = 40

BR = 2000
BE = 4000

_mesh = plsc.VectorSubcoreMesh(core_axis_name="c", subcore_axis_name="s")
_CP = pltpu.CompilerParams(needs_layout_passes=False)


def _delta_body(src_hbm, dst_hbm, s2_hbm, delta_hbm,
                es_m, ed_m, rows_d, rows_s, semd, sems):
    cid = lax.axis_index("c")
    sid = lax.axis_index("s")
    base = (cid * NS + sid) * E_TILE_D

    def _chunk(c, carry):
        off = base + c * ECD
        pltpu.sync_copy(src_hbm.at[pl.ds(off, ECD)], es_m)
        pltpu.sync_copy(dst_hbm.at[pl.ds(off, ECD)], ed_m)
        cpd = pltpu.async_copy(s2_hbm.at[ed_m], rows_d, semd)
        cps = pltpu.async_copy(s2_hbm.at[es_m], rows_s, sems)
        cpd.wait()
        cps.wait()
        iot = lax.iota(i32, L)

        def _sub(r, carry2):
            ridx = jnp.broadcast_to(r, (L,)).astype(i32)
            for cc in range(D // L):
                cidx = cc * L + iot
                vd = plsc.load_gather(rows_d, [ridx, cidx])
                vs = plsc.load_gather(rows_s, [ridx, cidx])
                plsc.store_scatter(rows_d, [ridx, cidx], vd - vs)
            return carry2

        lax.fori_loop(0, ECD, _sub, 0)
        pltpu.sync_copy(rows_d, delta_hbm.at[pl.ds(off, ECD)])
        return carry

    lax.fori_loop(0, NCH_D, _chunk, 0)


_sc_delta = functools.partial(
    pl.kernel,
    out_type=jax.ShapeDtypeStruct((E, D), f32),
    mesh=_mesh,
    compiler_params=_CP,
    scratch_types=[
        pltpu.VMEM((ECD,), i32),
        pltpu.VMEM((ECD,), i32),
        pltpu.VMEM((ECD, D), f32),
        pltpu.VMEM((ECD, D), f32),
        pltpu.SemaphoreType.DMA,
        pltpu.SemaphoreType.DMA,
    ],
)(_delta_body)


def _sc_body(src_hbm, dst_hbm, al_hbm, s2_hbm, agg_hbm, part_hbm,
             m_v, den_v, af_v, ed_v, acc_v, tmp_v,
             ds_s, vs_s, es_m, ed_m, ea_m, coef_v, rows_v, cs_v,
             red_sh, agg_sh, sem):
    cid = lax.axis_index("c")
    sid = lax.axis_index("s")
    iot = lax.iota(i32, L)
    zero16 = jnp.zeros((L,), f32)

    def _zm(k, carry):
        m_v[pl.ds(k * L, L)] = zero16
        den_v[pl.ds(k * L, L)] = zero16
        return carry

    lax.fori_loop(0, PN // L, _zm, 0)

    def _seg_combine(dv, val, op):
        sk, sv = plsc.sort_key_val(dv, val)
        ds_s[...] = sk
        vs_s[...] = sv
        v = sv
        for sh in (1, 2, 4, 8):
            jj = jnp.maximum(iot - sh, 0)
            pv = plsc.load_gather(vs_s, [jj])
            pd = plsc.load_gather(ds_s, [jj])
            take = (pd == sk) & (iot >= sh)
            v = jnp.where(take, op(v, pv), v)
            vs_s[...] = v
        nd = plsc.load_gather(ds_s, [jnp.minimum(iot + 1, L - 1)])
        last = (nd != sk) | (iot == L - 1)
        return sk, v, last

    ebase = sid * E_TILE_S

    def _max_chunk(c, carry):
        off = ebase + c * EC_S
        pltpu.sync_copy(al_hbm.at[pl.ds(off, EC_S)], af_v)
        pltpu.sync_copy(dst_hbm.at[pl.ds(off, EC_S)], ed_v)

        def _vr(j, carry2):
            av = af_v[pl.ds(j * L, L)]
            dv = ed_v[pl.ds(j * L, L)]
            sk, v, last = _seg_combine(dv, av, jnp.maximum)
            cur = plsc.load_gather(m_v, [sk])
            plsc.store_scatter(m_v, [sk], jnp.maximum(cur, v), mask=last)
            return carry2

        lax.fori_loop(0, NVR_S, _vr, 0)
        return carry

    lax.fori_loop(0, NCH_S, _max_chunk, 0)

    nbase = sid * CHN
    pltpu.sync_copy(m_v, part_hbm.at[cid, sid])
    plsc.subcore_barrier()
    pltpu.sync_copy(part_hbm.at[cid, 0, pl.ds(nbase, CHN)], acc_v)
    for k in range(1, NS):
        pltpu.sync_copy(part_hbm.at[cid, k, pl.ds(nbase, CHN)], tmp_v)

        def _redm(q, carry):
            acc_v[pl.ds(q * L, L)] = jnp.maximum(acc_v[pl.ds(q * L, L)],
                                                 tmp_v[pl.ds(q * L, L)])
            return carry

        lax.fori_loop(0, CHN // L, _redm, 0)
    pltpu.sync_copy(acc_v, red_sh.at[pl.ds(nbase, CHN)])
    plsc.subcore_barrier()
    pltpu.sync_copy(red_sh, m_v)

    def _den_chunk(c, carry):
        off = ebase + c * EC_S
        pltpu.sync_copy(al_hbm.at[pl.ds(off, EC_S)], af_v)
        pltpu.sync_copy(dst_hbm.at[pl.ds(off, EC_S)], ed_v)

        def _vr(j, carry2):
            av = af_v[pl.ds(j * L, L)]
            dv = ed_v[pl.ds(j * L, L)]
            w = jnp.exp(av - plsc.load_gather(m_v, [dv]))
            sk, v, last = _seg_combine(dv, w, jnp.add)
            cur = plsc.load_gather(den_v, [sk])
            plsc.store_scatter(den_v, [sk], cur + v, mask=last)
            return carry2

        lax.fori_loop(0, NVR_S, _vr, 0)
        return carry

    lax.fori_loop(0, NCH_S, _den_chunk, 0)

    pltpu.sync_copy(den_v, part_hbm.at[cid, sid])
    plsc.subcore_barrier()
    pltpu.sync_copy(part_hbm.at[cid, 0, pl.ds(nbase, CHN)], acc_v)
    for k in range(1, NS):
        pltpu.sync_copy(part_hbm.at[cid, k, pl.ds(nbase, CHN)], tmp_v)

        def _redd(q, carry):
            acc_v[pl.ds(q * L, L)] = (acc_v[pl.ds(q * L, L)]
                                      + tmp_v[pl.ds(q * L, L)])
            return carry

        lax.fori_loop(0, CHN // L, _redd, 0)

    def _selfterm(q, carry):
        mm = m_v[pl.ds(nbase + q * L, L)]
        acc_v[pl.ds(q * L, L)] = (acc_v[pl.ds(q * L, L)]
                                  + jnp.exp(-mm) + f32(1e-16))
        return carry

    lax.fori_loop(0, CHN // L, _selfterm, 0)
    pltpu.sync_copy(acc_v, red_sh.at[pl.ds(nbase, CHN)])
    plsc.subcore_barrier()
    pltpu.sync_copy(red_sh, den_v)

    csc = jnp.where(cid == 0, f32(1.0), f32(0.0))
    gbase = sid * CHN
    nself = jnp.where(gbase + CHN > N, (N - gbase) // RSELF, CHN // RSELF)

    def _csq(q, carry):
        mm = m_v[pl.ds(gbase + q * L, L)]
        dd = den_v[pl.ds(gbase + q * L, L)]
        cs_v[pl.ds(q * L, L)] = jnp.exp(-mm) / dd * csc
        return carry

    lax.fori_loop(0, CHN // L, _csq, 0)

    def _sinit(c, carry):
        g0 = gbase + c * RSELF
        pltpu.sync_copy(s2_hbm.at[pl.ds(g0, RSELF)], rows_v.at[pl.ds(0, RSELF)])
        for r in range(RSELF):
            ridx = jnp.broadcast_to(c * RSELF + r, (L,)).astype(i32)
            spl = plsc.load_gather(cs_v, [ridx])
            for cc in range(D // L):
                rows_v[r, pl.ds(cc * L, L)] = rows_v[r, pl.ds(cc * L, L)] * spl
        pltpu.sync_copy(rows_v.at[pl.ds(0, RSELF)], agg_sh.at[pl.ds(g0, RSELF)])
        return carry

    lax.fori_loop(0, nself, _sinit, 0)
    plsc.subcore_barrier()

    mbase = cid * E_HALF + sid * E_TILE_M

    def _msg(c, carry):
        off = mbase + c * ECM
        pltpu.sync_copy(src_hbm.at[pl.ds(off, ECM)], es_m)
        pltpu.sync_copy(dst_hbm.at[pl.ds(off, ECM)], ed_m)
        pltpu.sync_copy(al_hbm.at[pl.ds(off, ECM)], ea_m)

        def _cf(j, carry2):
            av = ea_m[pl.ds(j * L, L)]
            dv = ed_m[pl.ds(j * L, L)]
            w = jnp.exp(av - plsc.load_gather(m_v, [dv]))
            coef_v[pl.ds(j * L, L)] = w / plsc.load_gather(den_v, [dv])
            return carry2

        lax.fori_loop(0, ECM // L, _cf, 0)
        pltpu.async_copy(s2_hbm.at[es_m], rows_v, sem).wait()

        def _scale(r, carry2):
            ridx = jnp.broadcast_to(r, (L,)).astype(i32)
            spl = plsc.load_gather(coef_v, [ridx])
            for cc in range(D // L):
                cidx = cc * L + iot
                v = plsc.load_gather(rows_v, [ridx, cidx])
                plsc.store_scatter(rows_v, [ridx, cidx], v * spl)
            return carry2

        lax.fori_loop(0, ECM, _scale, 0)
        pltpu.sync_copy(rows_v, agg_sh.at[ed_m], add=True)
        return carry

    lax.fori_loop(0, NCH_M, _msg, 0)
    plsc.subcore_barrier()

    pltpu.sync_copy(agg_sh.at[pl.ds(gbase, CHN)],
                    agg_hbm.at[cid, pl.ds(gbase, CHN)])


_sc_layer = functools.partial(
    pl.kernel,
    out_type=(jax.ShapeDtypeStruct((NC, PN, D), f32),
              jax.ShapeDtypeStruct((NC, NS, PN), f32)),
    mesh=_mesh,
    compiler_params=_CP,
    scratch_types=[
        pltpu.VMEM((PN,), f32),
        pltpu.VMEM((PN,), f32),
        pltpu.VMEM((EC_S,), f32),
        pltpu.VMEM((EC_S,), i32),
        pltpu.VMEM((CHN,), f32),
        pltpu.VMEM((CHN,), f32),
        pltpu.VMEM((L,), i32),
        pltpu.VMEM((L,), f32),
        pltpu.VMEM((ECM,), i32),
        pltpu.VMEM((ECM,), i32),
        pltpu.VMEM((ECM,), f32),
        pltpu.VMEM((ECM,), f32),
        pltpu.VMEM((ECM, D), f32),
        pltpu.VMEM((CHN,), f32),
        pltpu.VMEM_SHARED((PN,), f32),
        pltpu.VMEM_SHARED((PN, D), f32),
        pltpu.SemaphoreType.DMA,
    ],
)(_sc_body)


def _s2_body(s_ref, w2_ref, b2_ref, s2_ref):
    s2_ref[...] = lax.dot_general(
        s_ref[...], w2_ref[...], (((1,), (1,)), ((), ())),
        preferred_element_type=f32) + b2_ref[...]


def _tc_s2(s, W2, b2):
    return pl.pallas_call(
        _s2_body,
        grid=(N // BR,),
        in_specs=[pl.BlockSpec((BR, D), lambda i: (i, 0)),
                  pl.BlockSpec((D, D), lambda i: (0, 0)),
                  pl.BlockSpec((1, D), lambda i: (0, 0))],
        out_specs=pl.BlockSpec((BR, D), lambda i: (i, 0)),
        out_shape=jax.ShapeDtypeStruct((N, D), f32),
    )(s, W2, b2.reshape(1, D))


def _al_body(d_ref, a_ref, o_ref):
    db = d_ref[...].astype(bf16)
    ab = a_ref[...].astype(bf16)
    o_ref[...] = lax.dot_general(db, ab, (((1,), (0,)), ((), ())),
                                 preferred_element_type=f32)


def _tc_alpha(delta, a):
    return pl.pallas_call(
        _al_body,
        grid=(E // BE,),
        in_specs=[pl.BlockSpec((BE, D), lambda i: (i, 0)),
                  pl.BlockSpec((D, 1), lambda i: (0, 0))],
        out_specs=pl.BlockSpec((BE, 1), lambda i: (i, 0)),
        out_shape=jax.ShapeDtypeStruct((E, 1), f32),
    )(delta, a)


def _out_body(s_ref, w1_ref, b1_ref, a0_ref, a1_ref, o_ref):
    o = lax.dot_general(s_ref[...], w1_ref[...], (((1,), (1,)), ((), ())),
                        preferred_element_type=f32) + b1_ref[...]
    o = o + a0_ref[0] + a1_ref[0]
    o_ref[...] = jnp.maximum(o, f32(0.0))


def _tc_out(s, W1, b1, agg):
    return pl.pallas_call(
        _out_body,
        grid=(N // BR,),
        in_specs=[pl.BlockSpec((BR, D), lambda i: (i, 0)),
                  pl.BlockSpec((D, D), lambda i: (0, 0)),
                  pl.BlockSpec((1, D), lambda i: (0, 0)),
                  pl.BlockSpec((1, BR, D), lambda i: (0, i, 0)),
                  pl.BlockSpec((1, BR, D), lambda i: (1, i, 0))],
        out_specs=pl.BlockSpec((BR, D), lambda i: (i, 0)),
        out_shape=jax.ShapeDtypeStruct((N, D), f32),
    )(s, W1, b1.reshape(1, D), agg, agg)


def kernel(s, edge_index,
           W1_0, b1_0, W2_0, b2_0, a_0,
           W1_1, b1_1, W2_1, b2_1, a_1,
           W1_2, b1_2, W2_2, b2_2, a_2):
    src = edge_index[0]
    dst = edge_index[1]
    params = [
        (W1_0, b1_0, W2_0, b2_0, a_0),
        (W1_1, b1_1, W2_1, b2_1, a_1),
        (W1_2, b1_2, W2_2, b2_2, a_2),
    ]
    out = s
    for (W1, b1, W2, b2, a) in params:
        s2 = _tc_s2(out, W2, b2)
        delta = _sc_delta(src, dst, s2)
        alpha = _tc_alpha(delta, a)
        agg, _ = _sc_layer(src, dst, alpha.reshape(E), s2)
        out = _tc_out(out, W1, b1, agg)
    return out


# speedup vs baseline: 4.4959x; 1.0754x over previous
import functools

import jax
import jax.numpy as jnp
from jax import lax
from jax.experimental import pallas as pl
from jax.experimental.pallas import tpu as pltpu, tpu_sc as plsc

f32 = jnp.float32
i32 = jnp.int32
bf16 = jnp.bfloat16

N = 10000
E = 320000
D = 128
L = 16
NC = 2
NS = 16
NW = NC * NS
PN = 10240
CHN = PN // NS

EC_S = 2000
E_TILE_S = E // NS
NCH_S = E_TILE_S // EC_S
NVR_S = EC_S // L

ECM = 80
E_HALF = E // NC
E_TILE_M = E_HALF // NS
NCH_M = E_TILE_M // ECM

E_TILE_D = E // NW
ECD = 400
NCH_D = E_TILE_D // ECD

RSELF = 40

BR = 2000
BE = 4000

_mesh = plsc.VectorSubcoreMesh(core_axis_name="c", subcore_axis_name="s")
_CP = pltpu.CompilerParams(needs_layout_passes=False)


def _delta_body(src_hbm, dst_hbm, s2_hbm, delta_hbm,
                es_m, ed_m, rows_d, rows_s, semd, sems):
    cid = lax.axis_index("c")
    sid = lax.axis_index("s")
    base = (cid * NS + sid) * E_TILE_D

    def _chunk(c, carry):
        off = base + c * ECD
        pltpu.sync_copy(src_hbm.at[pl.ds(off, ECD)], es_m)
        pltpu.sync_copy(dst_hbm.at[pl.ds(off, ECD)], ed_m)
        cpd = pltpu.async_copy(s2_hbm.at[ed_m], rows_d, semd)
        cps = pltpu.async_copy(s2_hbm.at[es_m], rows_s, sems)
        cpd.wait()
        cps.wait()
        iot = lax.iota(i32, L)

        def _sub(r, carry2):
            ridx = jnp.broadcast_to(r, (L,)).astype(i32)
            for cc in range(D // L):
                cidx = cc * L + iot
                vd = plsc.load_gather(rows_d, [ridx, cidx])
                vs = plsc.load_gather(rows_s, [ridx, cidx])
                plsc.store_scatter(rows_d, [ridx, cidx], vd - vs)
            return carry2

        lax.fori_loop(0, ECD, _sub, 0)
        pltpu.sync_copy(rows_d, delta_hbm.at[pl.ds(off, ECD)])
        return carry

    lax.fori_loop(0, NCH_D, _chunk, 0)


_sc_delta = functools.partial(
    pl.kernel,
    out_type=jax.ShapeDtypeStruct((E, D), f32),
    mesh=_mesh,
    compiler_params=_CP,
    scratch_types=[
        pltpu.VMEM((ECD,), i32),
        pltpu.VMEM((ECD,), i32),
        pltpu.VMEM((ECD, D), f32),
        pltpu.VMEM((ECD, D), f32),
        pltpu.SemaphoreType.DMA,
        pltpu.SemaphoreType.DMA,
    ],
)(_delta_body)


def _sc_body(src_hbm, dst_hbm, al_hbm, s2_hbm, agg_hbm, part_hbm,
             m_v, den_v, af_v, ed_v, acc_v, tmp_v,
             ds_s, vs_s, es_m, ed_m, ea_m, es_m2, ed_m2, ea_m2,
             coef_v, rows_v, rows_v2, cs_v,
             red_sh, agg_sh, sem, sem2):
    cid = lax.axis_index("c")
    sid = lax.axis_index("s")
    iot = lax.iota(i32, L)
    zero16 = jnp.zeros((L,), f32)

    def _zm(k, carry):
        m_v[pl.ds(k * L, L)] = zero16
        den_v[pl.ds(k * L, L)] = zero16
        return carry

    lax.fori_loop(0, PN // L, _zm, 0)

    def _seg_combine(dv, val, op):
        sk, sv = plsc.sort_key_val(dv, val)
        ds_s[...] = sk
        vs_s[...] = sv
        v = sv
        for sh in (1, 2, 4, 8):
            jj = jnp.maximum(iot - sh, 0)
            pv = plsc.load_gather(vs_s, [jj])
            pd = plsc.load_gather(ds_s, [jj])
            take = (pd == sk) & (iot >= sh)
            v = jnp.where(take, op(v, pv), v)
            vs_s[...] = v
        nd = plsc.load_gather(ds_s, [jnp.minimum(iot + 1, L - 1)])
        last = (nd != sk) | (iot == L - 1)
        return sk, v, last

    ebase = sid * E_TILE_S

    def _max_chunk(c, carry):
        off = ebase + c * EC_S
        pltpu.sync_copy(al_hbm.at[pl.ds(off, EC_S)], af_v)
        pltpu.sync_copy(dst_hbm.at[pl.ds(off, EC_S)], ed_v)

        def _vr(j, carry2):
            av = af_v[pl.ds(j * L, L)]
            dv = ed_v[pl.ds(j * L, L)]
            sk, v, last = _seg_combine(dv, av, jnp.maximum)
            cur = plsc.load_gather(m_v, [sk])
            plsc.store_scatter(m_v, [sk], jnp.maximum(cur, v), mask=last)
            return carry2

        lax.fori_loop(0, NVR_S, _vr, 0)
        return carry

    lax.fori_loop(0, NCH_S, _max_chunk, 0)

    nbase = sid * CHN
    pltpu.sync_copy(m_v, part_hbm.at[cid, sid])
    plsc.subcore_barrier()
    pltpu.sync_copy(part_hbm.at[cid, 0, pl.ds(nbase, CHN)], acc_v)
    for k in range(1, NS):
        pltpu.sync_copy(part_hbm.at[cid, k, pl.ds(nbase, CHN)], tmp_v)

        def _redm(q, carry):
            acc_v[pl.ds(q * L, L)] = jnp.maximum(acc_v[pl.ds(q * L, L)],
                                                 tmp_v[pl.ds(q * L, L)])
            return carry

        lax.fori_loop(0, CHN // L, _redm, 0)
    pltpu.sync_copy(acc_v, red_sh.at[pl.ds(nbase, CHN)])
    plsc.subcore_barrier()
    pltpu.sync_copy(red_sh, m_v)

    def _den_chunk(c, carry):
        off = ebase + c * EC_S
        pltpu.sync_copy(al_hbm.at[pl.ds(off, EC_S)], af_v)
        pltpu.sync_copy(dst_hbm.at[pl.ds(off, EC_S)], ed_v)

        def _vr(j, carry2):
            av = af_v[pl.ds(j * L, L)]
            dv = ed_v[pl.ds(j * L, L)]
            w = jnp.exp(av - plsc.load_gather(m_v, [dv]))
            sk, v, last = _seg_combine(dv, w, jnp.add)
            cur = plsc.load_gather(den_v, [sk])
            plsc.store_scatter(den_v, [sk], cur + v, mask=last)
            return carry2

        lax.fori_loop(0, NVR_S, _vr, 0)
        return carry

    lax.fori_loop(0, NCH_S, _den_chunk, 0)

    pltpu.sync_copy(den_v, part_hbm.at[cid, sid])
    plsc.subcore_barrier()
    pltpu.sync_copy(part_hbm.at[cid, 0, pl.ds(nbase, CHN)], acc_v)
    for k in range(1, NS):
        pltpu.sync_copy(part_hbm.at[cid, k, pl.ds(nbase, CHN)], tmp_v)

        def _redd(q, carry):
            acc_v[pl.ds(q * L, L)] = (acc_v[pl.ds(q * L, L)]
                                      + tmp_v[pl.ds(q * L, L)])
            return carry

        lax.fori_loop(0, CHN // L, _redd, 0)

    def _selfterm(q, carry):
        mm = m_v[pl.ds(nbase + q * L, L)]
        acc_v[pl.ds(q * L, L)] = (acc_v[pl.ds(q * L, L)]
                                  + jnp.exp(-mm) + f32(1e-16))
        return carry

    lax.fori_loop(0, CHN // L, _selfterm, 0)
    pltpu.sync_copy(acc_v, red_sh.at[pl.ds(nbase, CHN)])
    plsc.subcore_barrier()
    pltpu.sync_copy(red_sh, den_v)

    csc = jnp.where(cid == 0, f32(1.0), f32(0.0))
    gbase = sid * CHN
    nself = jnp.where(gbase + CHN > N, (N - gbase) // RSELF, CHN // RSELF)

    def _csq(q, carry):
        mm = m_v[pl.ds(gbase + q * L, L)]
        dd = den_v[pl.ds(gbase + q * L, L)]
        cs_v[pl.ds(q * L, L)] = jnp.exp(-mm) / dd * csc
        return carry

    lax.fori_loop(0, CHN // L, _csq, 0)

    def _sinit(c, carry):
        g0 = gbase + c * RSELF
        pltpu.sync_copy(s2_hbm.at[pl.ds(g0, RSELF)], rows_v.at[pl.ds(0, RSELF)])
        for r in range(RSELF):
            ridx = jnp.broadcast_to(c * RSELF + r, (L,)).astype(i32)
            spl = plsc.load_gather(cs_v, [ridx])
            for cc in range(D // L):
                rows_v[r, pl.ds(cc * L, L)] = rows_v[r, pl.ds(cc * L, L)] * spl
        pltpu.sync_copy(rows_v.at[pl.ds(0, RSELF)], agg_sh.at[pl.ds(g0, RSELF)])
        return carry

    lax.fori_loop(0, nself, _sinit, 0)
    plsc.subcore_barrier()

    mbase = cid * E_HALF + sid * E_TILE_M

    def _ldidx(c, es, ed, ea):
        off = mbase + c * ECM
        pltpu.sync_copy(src_hbm.at[pl.ds(off, ECM)], es)
        pltpu.sync_copy(dst_hbm.at[pl.ds(off, ECM)], ed)
        pltpu.sync_copy(al_hbm.at[pl.ds(off, ECM)], ea)

    def _wait(rows, sm):
        pltpu.make_async_copy(s2_hbm.at[pl.ds(0, ECM)], rows, sm).wait()

    def _consume(ed, ea, rows):
        def _cf(j, carry2):
            av = ea[pl.ds(j * L, L)]
            dv = ed[pl.ds(j * L, L)]
            w = jnp.exp(av - plsc.load_gather(m_v, [dv]))
            coef_v[pl.ds(j * L, L)] = w / plsc.load_gather(den_v, [dv])
            return carry2

        lax.fori_loop(0, ECM // L, _cf, 0)

        def _scale(r, carry2):
            ridx = jnp.broadcast_to(r, (L,)).astype(i32)
            spl = plsc.load_gather(coef_v, [ridx])
            for cc in range(D // L):
                cidx = cc * L + iot
                v = plsc.load_gather(rows, [ridx, cidx])
                plsc.store_scatter(rows, [ridx, cidx], v * spl)
            return carry2

        lax.fori_loop(0, ECM, _scale, 0)
        pltpu.sync_copy(rows, agg_sh.at[ed], add=True)

    _ldidx(0, es_m, ed_m, ea_m)
    pltpu.async_copy(s2_hbm.at[es_m], rows_v, sem)

    def _pair(i, carry):
        c = 2 * i
        _ldidx(c + 1, es_m2, ed_m2, ea_m2)
        pltpu.async_copy(s2_hbm.at[es_m2], rows_v2, sem2)
        _wait(rows_v, sem)
        _consume(ed_m, ea_m, rows_v)
        _ldidx(c + 2, es_m, ed_m, ea_m)
        pltpu.async_copy(s2_hbm.at[es_m], rows_v, sem)
        _wait(rows_v2, sem2)
        _consume(ed_m2, ea_m2, rows_v2)
        return carry

    lax.fori_loop(0, (NCH_M - 1) // 2, _pair, 0)
    _wait(rows_v, sem)
    _consume(ed_m, ea_m, rows_v)
    plsc.subcore_barrier()

    pltpu.sync_copy(agg_sh.at[pl.ds(gbase, CHN)],
                    agg_hbm.at[cid, pl.ds(gbase, CHN)])


_sc_layer = functools.partial(
    pl.kernel,
    out_type=(jax.ShapeDtypeStruct((NC, PN, D), f32),
              jax.ShapeDtypeStruct((NC, NS, PN), f32)),
    mesh=_mesh,
    compiler_params=_CP,
    scratch_types=[
        pltpu.VMEM((PN,), f32),
        pltpu.VMEM((PN,), f32),
        pltpu.VMEM((EC_S,), f32),
        pltpu.VMEM((EC_S,), i32),
        pltpu.VMEM((CHN,), f32),
        pltpu.VMEM((CHN,), f32),
        pltpu.VMEM((L,), i32),
        pltpu.VMEM((L,), f32),
        pltpu.VMEM((ECM,), i32),
        pltpu.VMEM((ECM,), i32),
        pltpu.VMEM((ECM,), f32),
        pltpu.VMEM((ECM,), i32),
        pltpu.VMEM((ECM,), i32),
        pltpu.VMEM((ECM,), f32),
        pltpu.VMEM((ECM,), f32),
        pltpu.VMEM((ECM, D), f32),
        pltpu.VMEM((ECM, D), f32),
        pltpu.VMEM((CHN,), f32),
        pltpu.VMEM_SHARED((PN,), f32),
        pltpu.VMEM_SHARED((PN, D), f32),
        pltpu.SemaphoreType.DMA,
        pltpu.SemaphoreType.DMA,
    ],
)(_sc_body)


def _s2_body(s_ref, w2_ref, b2_ref, s2_ref):
    s2_ref[...] = lax.dot_general(
        s_ref[...], w2_ref[...], (((1,), (1,)), ((), ())),
        preferred_element_type=f32) + b2_ref[...]


def _tc_s2(s, W2, b2):
    return pl.pallas_call(
        _s2_body,
        grid=(N // BR,),
        in_specs=[pl.BlockSpec((BR, D), lambda i: (i, 0)),
                  pl.BlockSpec((D, D), lambda i: (0, 0)),
                  pl.BlockSpec((1, D), lambda i: (0, 0))],
        out_specs=pl.BlockSpec((BR, D), lambda i: (i, 0)),
        out_shape=jax.ShapeDtypeStruct((N, D), f32),
    )(s, W2, b2.reshape(1, D))


def _al_body(d_ref, a_ref, o_ref):
    db = d_ref[...].astype(bf16)
    ab = a_ref[...].astype(bf16)
    o_ref[...] = lax.dot_general(db, ab, (((1,), (0,)), ((), ())),
                                 preferred_element_type=f32)


def _tc_alpha(delta, a):
    return pl.pallas_call(
        _al_body,
        grid=(E // BE,),
        in_specs=[pl.BlockSpec((BE, D), lambda i: (i, 0)),
                  pl.BlockSpec((D, 1), lambda i: (0, 0))],
        out_specs=pl.BlockSpec((BE, 1), lambda i: (i, 0)),
        out_shape=jax.ShapeDtypeStruct((E, 1), f32),
    )(delta, a)


def _out_body(s_ref, w1_ref, b1_ref, a0_ref, a1_ref, o_ref):
    o = lax.dot_general(s_ref[...], w1_ref[...], (((1,), (1,)), ((), ())),
                        preferred_element_type=f32) + b1_ref[...]
    o = o + a0_ref[0] + a1_ref[0]
    o_ref[...] = jnp.maximum(o, f32(0.0))


def _tc_out(s, W1, b1, agg):
    return pl.pallas_call(
        _out_body,
        grid=(N // BR,),
        in_specs=[pl.BlockSpec((BR, D), lambda i: (i, 0)),
                  pl.BlockSpec((D, D), lambda i: (0, 0)),
                  pl.BlockSpec((1, D), lambda i: (0, 0)),
                  pl.BlockSpec((1, BR, D), lambda i: (0, i, 0)),
                  pl.BlockSpec((1, BR, D), lambda i: (1, i, 0))],
        out_specs=pl.BlockSpec((BR, D), lambda i: (i, 0)),
        out_shape=jax.ShapeDtypeStruct((N, D), f32),
    )(s, W1, b1.reshape(1, D), agg, agg)


def kernel(s, edge_index,
           W1_0, b1_0, W2_0, b2_0, a_0,
           W1_1, b1_1, W2_1, b2_1, a_1,
           W1_2, b1_2, W2_2, b2_2, a_2):
    src = edge_index[0]
    dst = edge_index[1]
    params = [
        (W1_0, b1_0, W2_0, b2_0, a_0),
        (W1_1, b1_1, W2_1, b2_1, a_1),
        (W1_2, b1_2, W2_2, b2_2, a_2),
    ]
    out = s
    for (W1, b1, W2, b2, a) in params:
        s2 = _tc_s2(out, W2, b2)
        delta = _sc_delta(src, dst, s2)
        alpha = _tc_alpha(delta, a)
        agg, _ = _sc_layer(src, dst, alpha.reshape(E), s2)
        out = _tc_out(out, W1, b1, agg)
    return out
